# Initial kernel scaffold; baseline (speedup 1.0000x reference)
import jax
import jax.numpy as jnp
from jax import lax
from jax.experimental import pallas as pl
from jax.experimental.pallas import tpu as pltpu
from jax.experimental.pallas import tpu_sc as plsc

NN = 10000
EE = 320000
DIN = 70
HH = 128
GG = 256

NC = 2
NS = 16
NW = NC * NS
NP = 10240
RPS = NP // NS
EPW = 10240
EP = NW * EPW
CHUNK = 128
NCH = EPW // CHUNK
DEGW = 16

BM = 512
GRID = NP // BM

_MESH = plsc.VectorSubcoreMesh(core_axis_name="c", subcore_axis_name="s",
                               num_cores=NC, num_subcores=NS)
_PREC = lax.Precision.HIGHEST


def _sc_deg_body(dstr, zeros_d, ones_d, out, dst_v, ones_v, acc):
    cid = lax.axis_index("c")
    sid = lax.axis_index("s")
    wid = sid * NC + cid
    pltpu.sync_copy(zeros_d, acc.at[pl.ds(sid * RPS, RPS)])
    pltpu.sync_copy(ones_d, ones_v)
    pltpu.sync_copy(dstr.at[wid], dst_v)
    plsc.subcore_barrier()

    def body(j, c):
        pltpu.sync_copy(ones_v, acc.at[dst_v.at[j]], add=True)
        return c

    lax.fori_loop(0, NCH, body, 0)
    plsc.subcore_barrier()
    pltpu.sync_copy(acc.at[pl.ds(sid * RPS, RPS)],
                    out.at[cid, pl.ds(sid * RPS, RPS)])


_sc_deg = pl.kernel(
    _sc_deg_body,
    out_type=jax.ShapeDtypeStruct((NC, NP, DEGW), jnp.float32),
    mesh=_MESH,
    scratch_types=[
        pltpu.VMEM((NCH, CHUNK), jnp.int32),
        pltpu.VMEM((CHUNK, DEGW), jnp.float32),
        pltpu.VMEM_SHARED((NP, DEGW), jnp.float32),
    ],
)


def _sc_gs_body(table, srcr, dstr, zeros_h, out,
                src_v, dst_v, rows0, rows1, acc, sem0, sem1):
    cid = lax.axis_index("c")
    sid = lax.axis_index("s")
    wid = sid * NC + cid
    pltpu.sync_copy(zeros_h, acc.at[pl.ds(sid * RPS, RPS)])
    pltpu.sync_copy(srcr.at[wid], src_v)
    pltpu.sync_copy(dstr.at[wid], dst_v)
    plsc.subcore_barrier()

    rows = (rows0, rows1)
    sems = (sem0, sem1)
    pltpu.async_copy(table.at[src_v.at[0]], rows0, sem0)
    pltpu.async_copy(table.at[src_v.at[1]], rows1, sem1)

    def body(g, c):
        for b in range(2):
            j = g * 2 + b
            pltpu.make_async_copy(table.at[src_v.at[j]], rows[b],
                                  sems[b]).wait()
            pltpu.sync_copy(rows[b], acc.at[dst_v.at[j]], add=True)
            nxt = j + 2

            @pl.when(nxt < NCH)
            def _():
                pltpu.async_copy(table.at[src_v.at[nxt]], rows[b], sems[b])
        return c

    lax.fori_loop(0, NCH // 2, body, 0)
    plsc.subcore_barrier()
    pltpu.sync_copy(acc.at[pl.ds(sid * RPS, RPS)],
                    out.at[cid, pl.ds(sid * RPS, RPS)])


_sc_gs = pl.kernel(
    _sc_gs_body,
    out_type=jax.ShapeDtypeStruct((NC, NP, HH), jnp.float32),
    mesh=_MESH,
    scratch_types=[
        pltpu.VMEM((NCH, CHUNK), jnp.int32),
        pltpu.VMEM((NCH, CHUNK), jnp.int32),
        pltpu.VMEM((CHUNK, HH), jnp.float32),
        pltpu.VMEM((CHUNK, HH), jnp.float32),
        pltpu.VMEM_SHARED((NP, HH), jnp.float32),
        pltpu.SemaphoreType.DMA,
        pltpu.SemaphoreType.DMA,
    ],
)


def _dinv_block(dega_ref, degb_ref):
    return lax.rsqrt(1.0 + dega_ref[0][:, :1] + degb_ref[0][:, :1])


def _tc_a_body(x_ref, w0_ref, b0_ref, w1_ref, b1_ref, dega_ref, degb_ref,
               out_ref):
    dinv = _dinv_block(dega_ref, degb_ref)
    h0 = jnp.dot(x_ref[...], w0_ref[...], precision=_PREC,
                 preferred_element_type=jnp.float32) + b0_ref[...]
    out_ref[...] = dinv * (jnp.dot(h0, w1_ref[...], precision=_PREC,
                                   preferred_element_type=jnp.float32)
                           + b1_ref[...])


def _tc_b_body(sa_ref, sb_ref, ht_ref, dega_ref, degb_ref, w_ref, b_ref,
               out_ref):
    dinv = _dinv_block(dega_ref, degb_ref)
    t = jnp.maximum(dinv * (sa_ref[0] + sb_ref[0] + ht_ref[...]), 0.0)
    out_ref[...] = dinv * (jnp.dot(t, w_ref[...], precision=_PREC,
                                   preferred_element_type=jnp.float32)
                           + b_ref[...])


def _tc_c_body(sa_ref, sb_ref, ht_ref, dega_ref, degb_ref, batch_ref,
               out_ref):
    dinv = _dinv_block(dega_ref, degb_ref)
    agg = dinv * (sa_ref[0] + sb_ref[0] + ht_ref[...])
    onehot = (batch_ref[...] ==
              lax.broadcasted_iota(jnp.int32, (BM, GG), 1)).astype(jnp.float32)
    contrib = lax.dot_general(onehot, agg, (((0,), (0,)), ((), ())),
                              precision=_PREC,
                              preferred_element_type=jnp.float32)

    @pl.when(pl.program_id(0) == 0)
    def _():
        out_ref[...] = jnp.zeros_like(out_ref)

    out_ref[...] += contrib


_rows_spec = pl.BlockSpec((BM, HH), lambda i: (i, 0))
_sa_spec = pl.BlockSpec((1, BM, HH), lambda i: (0, i, 0))
_sb_spec = pl.BlockSpec((1, BM, HH), lambda i: (1, i, 0))
_dega_spec = pl.BlockSpec((1, BM, DEGW), lambda i: (0, i, 0))
_degb_spec = pl.BlockSpec((1, BM, DEGW), lambda i: (1, i, 0))
_w_spec = pl.BlockSpec((HH, HH), lambda i: (0, 0))
_b_spec = pl.BlockSpec((1, HH), lambda i: (0, 0))

_tc_a = pl.pallas_call(
    _tc_a_body,
    grid=(GRID,),
    in_specs=[_rows_spec, _w_spec, _b_spec, _w_spec, _b_spec,
              _dega_spec, _degb_spec],
    out_specs=_rows_spec,
    out_shape=jax.ShapeDtypeStruct((NP, HH), jnp.float32),
)

_tc_b = pl.pallas_call(
    _tc_b_body,
    grid=(GRID,),
    in_specs=[_sa_spec, _sb_spec, _rows_spec, _dega_spec, _degb_spec,
              _w_spec, _b_spec],
    out_specs=_rows_spec,
    out_shape=jax.ShapeDtypeStruct((NP, HH), jnp.float32),
)

_tc_c = pl.pallas_call(
    _tc_c_body,
    grid=(GRID,),
    in_specs=[_sa_spec, _sb_spec, _rows_spec, _dega_spec, _degb_spec,
              pl.BlockSpec((BM, 1), lambda i: (i, 0))],
    out_specs=pl.BlockSpec((GG, HH), lambda i: (0, 0)),
    out_shape=jax.ShapeDtypeStruct((GG, HH), jnp.float32),
)


def kernel(x, edge_index, batch, W0, b0, W1, b1, W2, b2, W3, b3):
    f32 = jnp.float32
    xp = jnp.zeros((NP, HH), f32).at[:NN, :DIN].set(x)
    w0p = jnp.zeros((HH, HH), f32).at[:DIN].set(W0)
    b0r = b0.reshape(1, HH)
    b1r = b1.reshape(1, HH)
    b2r = b2.reshape(1, HH)
    b3r = b3.reshape(1, HH)

    epad = EP - EE
    srcp = jnp.concatenate(
        [edge_index[0], jnp.full((epad,), NN, jnp.int32)]).reshape(
            NW, NCH, CHUNK)
    dstp = jnp.concatenate(
        [edge_index[1], jnp.full((epad,), NN, jnp.int32)]).reshape(
            NW, NCH, CHUNK)
    batchp = jnp.concatenate(
        [batch, jnp.full((NP - NN,), GG, jnp.int32)]).reshape(NP, 1)

    zeros_h = jnp.zeros((RPS, HH), f32)
    zeros_d = jnp.zeros((RPS, DEGW), f32)
    ones_d = jnp.ones((CHUNK, DEGW), f32)

    deg2 = _sc_deg(dstp, zeros_d, ones_d)
    ht1 = _tc_a(xp, w0p, b0r, W1, b1r, deg2, deg2)
    s1 = _sc_gs(ht1, srcp, dstp, zeros_h)
    ht2 = _tc_b(s1, s1, ht1, deg2, deg2, W2, b2r)
    s2 = _sc_gs(ht2, srcp, dstp, zeros_h)
    ht3 = _tc_b(s2, s2, ht2, deg2, deg2, W3, b3r)
    s3 = _sc_gs(ht3, srcp, dstp, zeros_h)
    pooled = _tc_c(s3, s3, ht3, deg2, deg2, batchp)
    return pooled


# speedup vs baseline: 9.8511x; 9.8511x over previous
import jax
import jax.numpy as jnp
from jax import lax
from jax.experimental import pallas as pl
from jax.experimental.pallas import tpu as pltpu
from jax.experimental.pallas import tpu_sc as plsc

NN = 10000
EE = 320000
DIN = 70
HH = 128
HC = HH // 2
GG = 256

NC = 2
NS = 16
NP = 10240
RPS = NP // NS
EPW = 20480
EP = NS * EPW
CHUNK = 128
NCH = EPW // CHUNK
DEGW = 16

BM = 512
GRID = NP // BM

_MESH = plsc.VectorSubcoreMesh(core_axis_name="c", subcore_axis_name="s",
                               num_cores=NC, num_subcores=NS)
_SC_PARAMS = pltpu.CompilerParams(use_tc_tiling_on_sc=False)
_PREC = lax.Precision.HIGHEST


def _sc_deg_body(dstr, zeros_d, ones_d, out, dst_v, ones_v, acc):
    cid = lax.axis_index("c")
    sid = lax.axis_index("s")
    pltpu.sync_copy(zeros_d, acc.at[pl.ds(sid * RPS, RPS)])
    pltpu.sync_copy(ones_d, ones_v)
    pltpu.sync_copy(dstr.at[sid], dst_v)
    plsc.subcore_barrier()

    def body(j, c):
        pltpu.sync_copy(ones_v, acc.at[dst_v.at[j]], add=True)
        return c

    lax.fori_loop(cid * (NCH // 2), (cid + 1) * (NCH // 2), body, 0)
    plsc.subcore_barrier()
    pltpu.sync_copy(acc.at[pl.ds(sid * RPS, RPS)],
                    out.at[cid, pl.ds(sid * RPS, RPS)])


_sc_deg = pl.kernel(
    _sc_deg_body,
    out_type=jax.ShapeDtypeStruct((NC, NP, DEGW), jnp.float32),
    mesh=_MESH,
    compiler_params=_SC_PARAMS,
    scratch_types=[
        pltpu.VMEM((NCH, CHUNK), jnp.int32),
        pltpu.VMEM((CHUNK, DEGW), jnp.float32),
        pltpu.VMEM_SHARED((NP, DEGW), jnp.float32),
    ],
)


def _sc_gs_body(t_lo, t_hi, srcr, dstr, zeros_c, out,
                src_v, dst_v, rows0, rows1, acc, sem0, sem1):
    cid = lax.axis_index("c")
    sid = lax.axis_index("s")
    pltpu.sync_copy(zeros_c, acc.at[pl.ds(sid * RPS, RPS)])
    pltpu.sync_copy(srcr.at[sid], src_v)
    pltpu.sync_copy(dstr.at[sid], dst_v)
    plsc.subcore_barrier()

    def run(table):
        rows = (rows0, rows1)
        sems = (sem0, sem1)
        pltpu.async_copy(table.at[src_v.at[0]], rows0, sem0)
        pltpu.async_copy(table.at[src_v.at[1]], rows1, sem1)

        def body(g, c):
            for b in range(2):
                j = g * 2 + b
                pltpu.make_async_copy(table.at[src_v.at[j]], rows[b],
                                      sems[b]).wait()
                pltpu.sync_copy(rows[b], acc.at[dst_v.at[j]], add=True)
                nxt = j + 2

                @pl.when(nxt < NCH)
                def _():
                    pltpu.async_copy(table.at[src_v.at[nxt]], rows[b],
                                     sems[b])
            return c

        lax.fori_loop(0, NCH // 2, body, 0)

    @pl.when(cid == 0)
    def _():
        run(t_lo)

    @pl.when(cid == 1)
    def _():
        run(t_hi)

    plsc.subcore_barrier()
    pltpu.sync_copy(acc.at[pl.ds(sid * RPS, RPS)],
                    out.at[cid, pl.ds(sid * RPS, RPS)])


_sc_gs = pl.kernel(
    _sc_gs_body,
    out_type=jax.ShapeDtypeStruct((NC, NP, HC), jnp.float32),
    mesh=_MESH,
    compiler_params=_SC_PARAMS,
    scratch_types=[
        pltpu.VMEM((NCH, CHUNK), jnp.int32),
        pltpu.VMEM((NCH, CHUNK), jnp.int32),
        pltpu.VMEM((CHUNK, HC), jnp.float32),
        pltpu.VMEM((CHUNK, HC), jnp.float32),
        pltpu.VMEM_SHARED((NP, HC), jnp.float32),
        pltpu.SemaphoreType.DMA,
        pltpu.SemaphoreType.DMA,
    ],
)


def _dinv_block(dega_ref, degb_ref):
    return lax.rsqrt(1.0 + dega_ref[0][:, :1] + degb_ref[0][:, :1])


def _split(res, lo_ref, hi_ref):
    lo_ref[...] = res[:, :HC]
    hi_ref[...] = res[:, HC:]


def _tc_a_body(x_ref, w0_ref, b0_ref, w1_ref, b1_ref, dega_ref, degb_ref,
               lo_ref, hi_ref):
    dinv = _dinv_block(dega_ref, degb_ref)
    h0 = jnp.dot(x_ref[...], w0_ref[...], precision=_PREC,
                 preferred_element_type=jnp.float32) + b0_ref[...]
    res = dinv * (jnp.dot(h0, w1_ref[...], precision=_PREC,
                          preferred_element_type=jnp.float32) + b1_ref[...])
    _split(res, lo_ref, hi_ref)


def _tc_b_body(sa_ref, sb_ref, tlo_ref, thi_ref, dega_ref, degb_ref,
               w_ref, b_ref, lo_ref, hi_ref):
    dinv = _dinv_block(dega_ref, degb_ref)
    s_plus_t = jnp.concatenate(
        [sa_ref[0] + tlo_ref[...], sb_ref[0] + thi_ref[...]], axis=1)
    t = jnp.maximum(dinv * s_plus_t, 0.0)
    res = dinv * (jnp.dot(t, w_ref[...], precision=_PREC,
                          preferred_element_type=jnp.float32) + b_ref[...])
    _split(res, lo_ref, hi_ref)


def _tc_c_body(sa_ref, sb_ref, tlo_ref, thi_ref, dega_ref, degb_ref,
               batch_ref, out_ref):
    dinv = _dinv_block(dega_ref, degb_ref)
    agg = dinv * jnp.concatenate(
        [sa_ref[0] + tlo_ref[...], sb_ref[0] + thi_ref[...]], axis=1)
    onehot = (batch_ref[...] ==
              lax.broadcasted_iota(jnp.int32, (BM, GG), 1)).astype(jnp.float32)
    contrib = lax.dot_general(onehot, agg, (((0,), (0,)), ((), ())),
                              precision=_PREC,
                              preferred_element_type=jnp.float32)

    @pl.when(pl.program_id(0) == 0)
    def _():
        out_ref[...] = jnp.zeros_like(out_ref)

    out_ref[...] += contrib


_x_spec = pl.BlockSpec((BM, HH), lambda i: (i, 0))
_half_spec = pl.BlockSpec((BM, HC), lambda i: (i, 0))
_sa_spec = pl.BlockSpec((1, BM, HC), lambda i: (0, i, 0))
_sb_spec = pl.BlockSpec((1, BM, HC), lambda i: (1, i, 0))
_dega_spec = pl.BlockSpec((1, BM, DEGW), lambda i: (0, i, 0))
_degb_spec = pl.BlockSpec((1, BM, DEGW), lambda i: (1, i, 0))
_w_spec = pl.BlockSpec((HH, HH), lambda i: (0, 0))
_b_spec = pl.BlockSpec((1, HH), lambda i: (0, 0))
_half_pair = (jax.ShapeDtypeStruct((NP, HC), jnp.float32),
              jax.ShapeDtypeStruct((NP, HC), jnp.float32))

_tc_a = pl.pallas_call(
    _tc_a_body,
    grid=(GRID,),
    in_specs=[_x_spec, _w_spec, _b_spec, _w_spec, _b_spec,
              _dega_spec, _degb_spec],
    out_specs=(_half_spec, _half_spec),
    out_shape=_half_pair,
)

_tc_b = pl.pallas_call(
    _tc_b_body,
    grid=(GRID,),
    in_specs=[_sa_spec, _sb_spec, _half_spec, _half_spec,
              _dega_spec, _degb_spec, _w_spec, _b_spec],
    out_specs=(_half_spec, _half_spec),
    out_shape=_half_pair,
)

_tc_c = pl.pallas_call(
    _tc_c_body,
    grid=(GRID,),
    in_specs=[_sa_spec, _sb_spec, _half_spec, _half_spec,
              _dega_spec, _degb_spec,
              pl.BlockSpec((BM, 1), lambda i: (i, 0))],
    out_specs=pl.BlockSpec((GG, HH), lambda i: (0, 0)),
    out_shape=jax.ShapeDtypeStruct((GG, HH), jnp.float32),
)


def kernel(x, edge_index, batch, W0, b0, W1, b1, W2, b2, W3, b3):
    f32 = jnp.float32
    xp = jnp.zeros((NP, HH), f32).at[:NN, :DIN].set(x)
    w0p = jnp.zeros((HH, HH), f32).at[:DIN].set(W0)
    b0r = b0.reshape(1, HH)
    b1r = b1.reshape(1, HH)
    b2r = b2.reshape(1, HH)
    b3r = b3.reshape(1, HH)

    epad = EP - EE
    srcp = jnp.concatenate(
        [edge_index[0], jnp.full((epad,), NN, jnp.int32)]).reshape(
            NS, NCH, CHUNK)
    dstp = jnp.concatenate(
        [edge_index[1], jnp.full((epad,), NN, jnp.int32)]).reshape(
            NS, NCH, CHUNK)
    batchp = jnp.concatenate(
        [batch, jnp.full((NP - NN,), GG, jnp.int32)]).reshape(NP, 1)

    zeros_c = jnp.zeros((RPS, HC), f32)
    zeros_d = jnp.zeros((RPS, DEGW), f32)
    ones_d = jnp.ones((CHUNK, DEGW), f32)

    deg2 = _sc_deg(dstp, zeros_d, ones_d)
    t1_lo, t1_hi = _tc_a(xp, w0p, b0r, W1, b1r, deg2, deg2)
    s1 = _sc_gs(t1_lo, t1_hi, srcp, dstp, zeros_c)
    t2_lo, t2_hi = _tc_b(s1, s1, t1_lo, t1_hi, deg2, deg2, W2, b2r)
    s2 = _sc_gs(t2_lo, t2_hi, srcp, dstp, zeros_c)
    t3_lo, t3_hi = _tc_b(s2, s2, t2_lo, t2_hi, deg2, deg2, W3, b3r)
    s3 = _sc_gs(t3_lo, t3_hi, srcp, dstp, zeros_c)
    pooled = _tc_c(s3, s3, t3_lo, t3_hi, deg2, deg2, batchp)
    return pooled


# speedup vs baseline: 9.8973x; 1.0047x over previous
import jax
import jax.numpy as jnp
from jax import lax
from jax.experimental import pallas as pl
from jax.experimental.pallas import tpu as pltpu
from jax.experimental.pallas import tpu_sc as plsc

NN = 10000
EE = 320000
DIN = 70
HH = 128
HC = HH // 2
GG = 256

NC = 2
NS = 16
NP = 10240
RPS = NP // NS
EPW = 20480
EP = NS * EPW
CHUNK = 128
NCH = EPW // CHUNK
DEGW = 16

BM = 512
GRID = NP // BM

_MESH = plsc.VectorSubcoreMesh(core_axis_name="c", subcore_axis_name="s",
                               num_cores=NC, num_subcores=NS)
_SC_PARAMS = pltpu.CompilerParams(use_tc_tiling_on_sc=False)
_PREC = lax.Precision.HIGHEST


def _sc_deg_body(dstr, zeros_d, ones_d, out, dst_v, ones_v, acc):
    cid = lax.axis_index("c")
    sid = lax.axis_index("s")
    pltpu.sync_copy(zeros_d, acc.at[pl.ds(sid * RPS, RPS)])
    pltpu.sync_copy(ones_d, ones_v)
    pltpu.sync_copy(dstr.at[sid], dst_v)
    plsc.subcore_barrier()

    def body(j, c):
        pltpu.sync_copy(ones_v, acc.at[dst_v.at[j]], add=True)
        return c

    lax.fori_loop(cid * (NCH // 2), (cid + 1) * (NCH // 2), body, 0)
    plsc.subcore_barrier()
    pltpu.sync_copy(acc.at[pl.ds(sid * RPS, RPS)],
                    out.at[cid, pl.ds(sid * RPS, RPS)])


_sc_deg = pl.kernel(
    _sc_deg_body,
    out_type=jax.ShapeDtypeStruct((NC, NP, DEGW), jnp.float32),
    mesh=_MESH,
    compiler_params=_SC_PARAMS,
    scratch_types=[
        pltpu.VMEM((NCH, CHUNK), jnp.int32),
        pltpu.VMEM((CHUNK, DEGW), jnp.float32),
        pltpu.VMEM_SHARED((NP, DEGW), jnp.float32),
    ],
)


KB = 4


def _sc_gs_body(t_lo, t_hi, srcr, dstr, zeros_c, out, *scratch):
    src_v, dst_v = scratch[0], scratch[1]
    rows = scratch[2:2 + KB]
    acc = scratch[2 + KB]
    gsems = scratch[3 + KB:3 + 2 * KB]
    ssems = scratch[3 + 2 * KB:3 + 3 * KB]
    cid = lax.axis_index("c")
    sid = lax.axis_index("s")
    pltpu.sync_copy(zeros_c, acc.at[pl.ds(sid * RPS, RPS)])
    pltpu.sync_copy(srcr.at[sid], src_v)
    pltpu.sync_copy(dstr.at[sid], dst_v)
    plsc.subcore_barrier()

    def run(table):
        for b in range(KB):
            pltpu.async_copy(table.at[src_v.at[b]], rows[b], gsems[b])

        def body(r, c):
            base = r * KB
            for b in range(KB):
                j = base + b
                pltpu.make_async_copy(table.at[src_v.at[j]], rows[b],
                                      gsems[b]).wait()
                pltpu.async_copy(rows[b], acc.at[dst_v.at[j]], ssems[b],
                                 add=True)
            for b in range(KB):
                pltpu.make_async_copy(rows[b], acc.at[dst_v.at[base + b]],
                                      ssems[b]).wait()
                nxt = base + KB + b

                @pl.when(nxt < NCH)
                def _():
                    pltpu.async_copy(table.at[src_v.at[nxt]], rows[b],
                                     gsems[b])
            return c

        lax.fori_loop(0, NCH // KB, body, 0)

    @pl.when(cid == 0)
    def _():
        run(t_lo)

    @pl.when(cid == 1)
    def _():
        run(t_hi)

    plsc.subcore_barrier()
    pltpu.sync_copy(acc.at[pl.ds(sid * RPS, RPS)],
                    out.at[cid, pl.ds(sid * RPS, RPS)])


_sc_gs = pl.kernel(
    _sc_gs_body,
    out_type=jax.ShapeDtypeStruct((NC, NP, HC), jnp.float32),
    mesh=_MESH,
    compiler_params=_SC_PARAMS,
    scratch_types=(
        [pltpu.VMEM((NCH, CHUNK), jnp.int32),
         pltpu.VMEM((NCH, CHUNK), jnp.int32)]
        + [pltpu.VMEM((CHUNK, HC), jnp.float32) for _ in range(KB)]
        + [pltpu.VMEM_SHARED((NP, HC), jnp.float32)]
        + [pltpu.SemaphoreType.DMA for _ in range(2 * KB)]
    ),
)


def _dinv_block(dega_ref, degb_ref):
    return lax.rsqrt(1.0 + dega_ref[0][:, :1] + degb_ref[0][:, :1])


def _split(res, lo_ref, hi_ref):
    lo_ref[...] = res[:, :HC]
    hi_ref[...] = res[:, HC:]


def _tc_a_body(x_ref, w0_ref, b0_ref, w1_ref, b1_ref, dega_ref, degb_ref,
               lo_ref, hi_ref):
    dinv = _dinv_block(dega_ref, degb_ref)
    h0 = jnp.dot(x_ref[...], w0_ref[...], precision=_PREC,
                 preferred_element_type=jnp.float32) + b0_ref[...]
    res = dinv * (jnp.dot(h0, w1_ref[...], precision=_PREC,
                          preferred_element_type=jnp.float32) + b1_ref[...])
    _split(res, lo_ref, hi_ref)


def _tc_b_body(sa_ref, sb_ref, tlo_ref, thi_ref, dega_ref, degb_ref,
               w_ref, b_ref, lo_ref, hi_ref):
    dinv = _dinv_block(dega_ref, degb_ref)
    s_plus_t = jnp.concatenate(
        [sa_ref[0] + tlo_ref[...], sb_ref[0] + thi_ref[...]], axis=1)
    t = jnp.maximum(dinv * s_plus_t, 0.0)
    res = dinv * (jnp.dot(t, w_ref[...], precision=_PREC,
                          preferred_element_type=jnp.float32) + b_ref[...])
    _split(res, lo_ref, hi_ref)


def _tc_c_body(sa_ref, sb_ref, tlo_ref, thi_ref, dega_ref, degb_ref,
               batch_ref, out_ref):
    dinv = _dinv_block(dega_ref, degb_ref)
    agg = dinv * jnp.concatenate(
        [sa_ref[0] + tlo_ref[...], sb_ref[0] + thi_ref[...]], axis=1)
    onehot = (batch_ref[...] ==
              lax.broadcasted_iota(jnp.int32, (BM, GG), 1)).astype(jnp.float32)
    contrib = lax.dot_general(onehot, agg, (((0,), (0,)), ((), ())),
                              precision=_PREC,
                              preferred_element_type=jnp.float32)

    @pl.when(pl.program_id(0) == 0)
    def _():
        out_ref[...] = jnp.zeros_like(out_ref)

    out_ref[...] += contrib


_x_spec = pl.BlockSpec((BM, HH), lambda i: (i, 0))
_half_spec = pl.BlockSpec((BM, HC), lambda i: (i, 0))
_sa_spec = pl.BlockSpec((1, BM, HC), lambda i: (0, i, 0))
_sb_spec = pl.BlockSpec((1, BM, HC), lambda i: (1, i, 0))
_dega_spec = pl.BlockSpec((1, BM, DEGW), lambda i: (0, i, 0))
_degb_spec = pl.BlockSpec((1, BM, DEGW), lambda i: (1, i, 0))
_w_spec = pl.BlockSpec((HH, HH), lambda i: (0, 0))
_b_spec = pl.BlockSpec((1, HH), lambda i: (0, 0))
_half_pair = (jax.ShapeDtypeStruct((NP, HC), jnp.float32),
              jax.ShapeDtypeStruct((NP, HC), jnp.float32))

_tc_a = pl.pallas_call(
    _tc_a_body,
    grid=(GRID,),
    in_specs=[_x_spec, _w_spec, _b_spec, _w_spec, _b_spec,
              _dega_spec, _degb_spec],
    out_specs=(_half_spec, _half_spec),
    out_shape=_half_pair,
)

_tc_b = pl.pallas_call(
    _tc_b_body,
    grid=(GRID,),
    in_specs=[_sa_spec, _sb_spec, _half_spec, _half_spec,
              _dega_spec, _degb_spec, _w_spec, _b_spec],
    out_specs=(_half_spec, _half_spec),
    out_shape=_half_pair,
)

_tc_c = pl.pallas_call(
    _tc_c_body,
    grid=(GRID,),
    in_specs=[_sa_spec, _sb_spec, _half_spec, _half_spec,
              _dega_spec, _degb_spec,
              pl.BlockSpec((BM, 1), lambda i: (i, 0))],
    out_specs=pl.BlockSpec((GG, HH), lambda i: (0, 0)),
    out_shape=jax.ShapeDtypeStruct((GG, HH), jnp.float32),
)


def kernel(x, edge_index, batch, W0, b0, W1, b1, W2, b2, W3, b3):
    f32 = jnp.float32
    xp = jnp.zeros((NP, HH), f32).at[:NN, :DIN].set(x)
    w0p = jnp.zeros((HH, HH), f32).at[:DIN].set(W0)
    b0r = b0.reshape(1, HH)
    b1r = b1.reshape(1, HH)
    b2r = b2.reshape(1, HH)
    b3r = b3.reshape(1, HH)

    epad = EP - EE
    srcp = jnp.concatenate(
        [edge_index[0], jnp.full((epad,), NN, jnp.int32)]).reshape(
            NS, NCH, CHUNK)
    dstp = jnp.concatenate(
        [edge_index[1], jnp.full((epad,), NN, jnp.int32)]).reshape(
            NS, NCH, CHUNK)
    batchp = jnp.concatenate(
        [batch, jnp.full((NP - NN,), GG, jnp.int32)]).reshape(NP, 1)

    zeros_c = jnp.zeros((RPS, HC), f32)
    zeros_d = jnp.zeros((RPS, DEGW), f32)
    ones_d = jnp.ones((CHUNK, DEGW), f32)

    deg2 = _sc_deg(dstp, zeros_d, ones_d)
    t1_lo, t1_hi = _tc_a(xp, w0p, b0r, W1, b1r, deg2, deg2)
    s1 = _sc_gs(t1_lo, t1_hi, srcp, dstp, zeros_c)
    t2_lo, t2_hi = _tc_b(s1, s1, t1_lo, t1_hi, deg2, deg2, W2, b2r)
    s2 = _sc_gs(t2_lo, t2_hi, srcp, dstp, zeros_c)
    t3_lo, t3_hi = _tc_b(s2, s2, t2_lo, t2_hi, deg2, deg2, W3, b3r)
    s3 = _sc_gs(t3_lo, t3_hi, srcp, dstp, zeros_c)
    pooled = _tc_c(s3, s3, t3_lo, t3_hi, deg2, deg2, batchp)
    return pooled


# speedup vs baseline: 15.1954x; 1.5353x over previous
import jax
import jax.numpy as jnp
from jax import lax
from jax.experimental import pallas as pl
from jax.experimental.pallas import tpu as pltpu
from jax.experimental.pallas import tpu_sc as plsc

NN = 10000
EE = 320000
DIN = 70
HH = 128
HC = HH // 2
GG = 256

NC = 2
NS = 16
NP = 10240
RPS = NP // NS
EPW = 20480
EP = NS * EPW
CHUNK = 128
NCH = EPW // CHUNK
DEGW = 16

BM = 512
GRID = NP // BM

_MESH = plsc.VectorSubcoreMesh(core_axis_name="c", subcore_axis_name="s",
                               num_cores=NC, num_subcores=NS)
_SC_PARAMS = pltpu.CompilerParams(use_tc_tiling_on_sc=False)
_PREC = lax.Precision.HIGHEST


def _sc_deg_body(dstr, zeros_d, ones_d, out, dst_v, ones_v, acc):
    cid = lax.axis_index("c")
    sid = lax.axis_index("s")
    pltpu.sync_copy(zeros_d, acc.at[pl.ds(sid * RPS, RPS)])
    pltpu.sync_copy(ones_d, ones_v)
    pltpu.sync_copy(dstr.at[sid], dst_v)
    plsc.subcore_barrier()

    def body(j, c):
        pltpu.sync_copy(ones_v, acc.at[dst_v.at[j]], add=True)
        return c

    lax.fori_loop(cid * (NCH // 2), (cid + 1) * (NCH // 2), body, 0)
    plsc.subcore_barrier()
    pltpu.sync_copy(acc.at[pl.ds(sid * RPS, RPS)],
                    out.at[cid, pl.ds(sid * RPS, RPS)])


_sc_deg = pl.kernel(
    _sc_deg_body,
    out_type=jax.ShapeDtypeStruct((NC, NP, DEGW), jnp.float32),
    mesh=_MESH,
    compiler_params=_SC_PARAMS,
    scratch_types=[
        pltpu.VMEM((NCH, CHUNK), jnp.int32),
        pltpu.VMEM((CHUNK, DEGW), jnp.float32),
        pltpu.VMEM_SHARED((NP, DEGW), jnp.float32),
    ],
)


KB = 4


NR = NCH // KB


def _sc_gs_body(t_lo, t_hi, srcr, dstr, zeros_c, out, *scratch):
    sidx, didx = scratch[0], scratch[1]
    rows = scratch[2:2 + KB]
    tsh = scratch[2 + KB]
    acc = scratch[3 + KB]
    gsems = scratch[4 + KB:4 + 2 * KB]
    ssems = scratch[4 + 2 * KB:4 + 3 * KB]
    isems = scratch[4 + 3 * KB:6 + 3 * KB]
    cid = lax.axis_index("c")
    sid = lax.axis_index("s")

    pltpu.sync_copy(zeros_c, acc.at[pl.ds(sid * RPS, RPS)])

    @pl.when(cid == 0)
    def _():
        pltpu.sync_copy(t_lo.at[pl.ds(sid * RPS, RPS)],
                        tsh.at[pl.ds(sid * RPS, RPS)])

    @pl.when(cid == 1)
    def _():
        pltpu.sync_copy(t_hi.at[pl.ds(sid * RPS, RPS)],
                        tsh.at[pl.ds(sid * RPS, RPS)])

    plsc.subcore_barrier()

    def prefetch(r, slot):
        pltpu.async_copy(srcr.at[sid, pl.ds(r * KB, KB)], sidx.at[slot],
                         isems[slot])
        pltpu.async_copy(dstr.at[sid, pl.ds(r * KB, KB)], didx.at[slot],
                         isems[slot])

    def wait_idx(r, slot):
        pltpu.make_async_copy(srcr.at[sid, pl.ds(r * KB, KB)],
                              sidx.at[slot], isems[slot]).wait()
        pltpu.make_async_copy(dstr.at[sid, pl.ds(r * KB, KB)],
                              didx.at[slot], isems[slot]).wait()

    prefetch(0, 0)

    def body(q, c):
        for slot in range(2):
            r = q * 2 + slot
            wait_idx(r, slot)

            @pl.when(r + 1 < NR)
            def _():
                prefetch(r + 1, 1 - slot)

            for b in range(KB):
                pltpu.async_copy(tsh.at[sidx.at[slot, b]], rows[b],
                                 gsems[b])
            for b in range(KB):
                pltpu.make_async_copy(tsh.at[sidx.at[slot, b]], rows[b],
                                      gsems[b]).wait()
                pltpu.async_copy(rows[b], acc.at[didx.at[slot, b]],
                                 ssems[b], add=True)
            for b in range(KB):
                pltpu.make_async_copy(rows[b], acc.at[didx.at[slot, b]],
                                      ssems[b]).wait()
        return c

    lax.fori_loop(0, NR // 2, body, 0)
    plsc.subcore_barrier()
    pltpu.sync_copy(acc.at[pl.ds(sid * RPS, RPS)],
                    out.at[cid, pl.ds(sid * RPS, RPS)])


_sc_gs = pl.kernel(
    _sc_gs_body,
    out_type=jax.ShapeDtypeStruct((NC, NP, HC), jnp.float32),
    mesh=_MESH,
    compiler_params=_SC_PARAMS,
    scratch_types=(
        [pltpu.VMEM((2, KB, CHUNK), jnp.int32),
         pltpu.VMEM((2, KB, CHUNK), jnp.int32)]
        + [pltpu.VMEM((CHUNK, HC), jnp.float32) for _ in range(KB)]
        + [pltpu.VMEM_SHARED((NP, HC), jnp.float32),
           pltpu.VMEM_SHARED((NP, HC), jnp.float32)]
        + [pltpu.SemaphoreType.DMA for _ in range(2 * KB + 2)]
    ),
)


def _dinv_block(dega_ref, degb_ref):
    return lax.rsqrt(1.0 + dega_ref[0][:, :1] + degb_ref[0][:, :1])


def _split(res, lo_ref, hi_ref):
    lo_ref[...] = res[:, :HC]
    hi_ref[...] = res[:, HC:]


def _tc_a_body(x_ref, w0_ref, b0_ref, w1_ref, b1_ref, dega_ref, degb_ref,
               lo_ref, hi_ref):
    dinv = _dinv_block(dega_ref, degb_ref)
    h0 = jnp.dot(x_ref[...], w0_ref[...], precision=_PREC,
                 preferred_element_type=jnp.float32) + b0_ref[...]
    res = dinv * (jnp.dot(h0, w1_ref[...], precision=_PREC,
                          preferred_element_type=jnp.float32) + b1_ref[...])
    _split(res, lo_ref, hi_ref)


def _tc_b_body(sa_ref, sb_ref, tlo_ref, thi_ref, dega_ref, degb_ref,
               w_ref, b_ref, lo_ref, hi_ref):
    dinv = _dinv_block(dega_ref, degb_ref)
    s_plus_t = jnp.concatenate(
        [sa_ref[0] + tlo_ref[...], sb_ref[0] + thi_ref[...]], axis=1)
    t = jnp.maximum(dinv * s_plus_t, 0.0)
    res = dinv * (jnp.dot(t, w_ref[...], precision=_PREC,
                          preferred_element_type=jnp.float32) + b_ref[...])
    _split(res, lo_ref, hi_ref)


def _tc_c_body(sa_ref, sb_ref, tlo_ref, thi_ref, dega_ref, degb_ref,
               batch_ref, out_ref):
    dinv = _dinv_block(dega_ref, degb_ref)
    agg = dinv * jnp.concatenate(
        [sa_ref[0] + tlo_ref[...], sb_ref[0] + thi_ref[...]], axis=1)
    onehot = (batch_ref[...] ==
              lax.broadcasted_iota(jnp.int32, (BM, GG), 1)).astype(jnp.float32)
    contrib = lax.dot_general(onehot, agg, (((0,), (0,)), ((), ())),
                              precision=_PREC,
                              preferred_element_type=jnp.float32)

    @pl.when(pl.program_id(0) == 0)
    def _():
        out_ref[...] = jnp.zeros_like(out_ref)

    out_ref[...] += contrib


_x_spec = pl.BlockSpec((BM, HH), lambda i: (i, 0))
_half_spec = pl.BlockSpec((BM, HC), lambda i: (i, 0))
_sa_spec = pl.BlockSpec((1, BM, HC), lambda i: (0, i, 0))
_sb_spec = pl.BlockSpec((1, BM, HC), lambda i: (1, i, 0))
_dega_spec = pl.BlockSpec((1, BM, DEGW), lambda i: (0, i, 0))
_degb_spec = pl.BlockSpec((1, BM, DEGW), lambda i: (1, i, 0))
_w_spec = pl.BlockSpec((HH, HH), lambda i: (0, 0))
_b_spec = pl.BlockSpec((1, HH), lambda i: (0, 0))
_half_pair = (jax.ShapeDtypeStruct((NP, HC), jnp.float32),
              jax.ShapeDtypeStruct((NP, HC), jnp.float32))

_tc_a = pl.pallas_call(
    _tc_a_body,
    grid=(GRID,),
    in_specs=[_x_spec, _w_spec, _b_spec, _w_spec, _b_spec,
              _dega_spec, _degb_spec],
    out_specs=(_half_spec, _half_spec),
    out_shape=_half_pair,
)

_tc_b = pl.pallas_call(
    _tc_b_body,
    grid=(GRID,),
    in_specs=[_sa_spec, _sb_spec, _half_spec, _half_spec,
              _dega_spec, _degb_spec, _w_spec, _b_spec],
    out_specs=(_half_spec, _half_spec),
    out_shape=_half_pair,
)

_tc_c = pl.pallas_call(
    _tc_c_body,
    grid=(GRID,),
    in_specs=[_sa_spec, _sb_spec, _half_spec, _half_spec,
              _dega_spec, _degb_spec,
              pl.BlockSpec((BM, 1), lambda i: (i, 0))],
    out_specs=pl.BlockSpec((GG, HH), lambda i: (0, 0)),
    out_shape=jax.ShapeDtypeStruct((GG, HH), jnp.float32),
)


def kernel(x, edge_index, batch, W0, b0, W1, b1, W2, b2, W3, b3):
    f32 = jnp.float32
    xp = jnp.zeros((NP, HH), f32).at[:NN, :DIN].set(x)
    w0p = jnp.zeros((HH, HH), f32).at[:DIN].set(W0)
    b0r = b0.reshape(1, HH)
    b1r = b1.reshape(1, HH)
    b2r = b2.reshape(1, HH)
    b3r = b3.reshape(1, HH)

    epad = EP - EE
    srcp = jnp.concatenate(
        [edge_index[0], jnp.full((epad,), NN, jnp.int32)]).reshape(
            NS, NCH, CHUNK)
    dstp = jnp.concatenate(
        [edge_index[1], jnp.full((epad,), NN, jnp.int32)]).reshape(
            NS, NCH, CHUNK)
    batchp = jnp.concatenate(
        [batch, jnp.full((NP - NN,), GG, jnp.int32)]).reshape(NP, 1)

    zeros_c = jnp.zeros((RPS, HC), f32)
    zeros_d = jnp.zeros((RPS, DEGW), f32)
    ones_d = jnp.ones((CHUNK, DEGW), f32)

    deg2 = _sc_deg(dstp, zeros_d, ones_d)
    t1_lo, t1_hi = _tc_a(xp, w0p, b0r, W1, b1r, deg2, deg2)
    s1 = _sc_gs(t1_lo, t1_hi, srcp, dstp, zeros_c)
    t2_lo, t2_hi = _tc_b(s1, s1, t1_lo, t1_hi, deg2, deg2, W2, b2r)
    s2 = _sc_gs(t2_lo, t2_hi, srcp, dstp, zeros_c)
    t3_lo, t3_hi = _tc_b(s2, s2, t2_lo, t2_hi, deg2, deg2, W3, b3r)
    s3 = _sc_gs(t3_lo, t3_hi, srcp, dstp, zeros_c)
    pooled = _tc_c(s3, s3, t3_lo, t3_hi, deg2, deg2, batchp)
    return pooled


# speedup vs baseline: 16.6897x; 1.0983x over previous
import jax
import jax.numpy as jnp
from jax import lax
from jax.experimental import pallas as pl
from jax.experimental.pallas import tpu as pltpu
from jax.experimental.pallas import tpu_sc as plsc

NN = 10000
EE = 320000
DIN = 70
HH = 128
HC = HH // 2
GG = 256

NC = 2
NS = 16
NP = 10240
RPS = NP // NS
EPW = 20480
EP = NS * EPW
CHUNK = 128
NCH = EPW // CHUNK
KB2 = 2
NR = NCH // KB2
DEGW = 16

BM = 1280
GRID = NP // BM

_MESH = plsc.VectorSubcoreMesh(core_axis_name="c", subcore_axis_name="s",
                               num_cores=NC, num_subcores=NS)
_SC_PARAMS = pltpu.CompilerParams(use_tc_tiling_on_sc=False)
_PREC = lax.Precision.HIGHEST


def _sc_deg_body(dstr, zeros_d, ones_d, out, dst_v, ones_v, acc):
    cid = lax.axis_index("c")
    sid = lax.axis_index("s")
    pltpu.sync_copy(zeros_d, acc.at[pl.ds(sid * RPS, RPS)])
    pltpu.sync_copy(ones_d, ones_v)
    pltpu.sync_copy(dstr.at[sid], dst_v)
    plsc.subcore_barrier()

    def body(j, c):
        pltpu.sync_copy(ones_v, acc.at[dst_v.at[j]], add=True)
        return c

    lax.fori_loop(cid * (NCH // 2), (cid + 1) * (NCH // 2), body, 0)
    plsc.subcore_barrier()
    pltpu.sync_copy(acc.at[pl.ds(sid * RPS, RPS)],
                    out.at[cid, pl.ds(sid * RPS, RPS)])


_sc_deg = pl.kernel(
    _sc_deg_body,
    out_type=jax.ShapeDtypeStruct((NC, NP, DEGW), jnp.float32),
    mesh=_MESH,
    compiler_params=_SC_PARAMS,
    scratch_types=[
        pltpu.VMEM((NCH, CHUNK), jnp.int32),
        pltpu.VMEM((CHUNK, DEGW), jnp.float32),
        pltpu.VMEM_SHARED((NP, DEGW), jnp.float32),
    ],
)


def _sc_gs_body(ht, srcr, dstr, zeros_c, out, *scratch):
    sidx, didx = scratch[0], scratch[1]
    rows = (scratch[2:4], scratch[4:6])
    tsh = scratch[6]
    acc = scratch[7]
    gsems = (scratch[8:10], scratch[10:12])
    ssems = (scratch[12:14], scratch[14:16])
    isems = scratch[16:20]
    cid = lax.axis_index("c")
    sid = lax.axis_index("s")

    pltpu.sync_copy(zeros_c, acc.at[pl.ds(sid * RPS, RPS)])

    @pl.when(cid == 0)
    def _():
        pltpu.sync_copy(ht.at[pl.ds(sid * RPS, RPS), pl.ds(0, HC)],
                        tsh.at[pl.ds(sid * RPS, RPS)])

    @pl.when(cid == 1)
    def _():
        pltpu.sync_copy(ht.at[pl.ds(sid * RPS, RPS), pl.ds(HC, HC)],
                        tsh.at[pl.ds(sid * RPS, RPS)])

    plsc.subcore_barrier()

    def prefetch(r, slot):
        pltpu.async_copy(srcr.at[sid, pl.ds(r * KB2, KB2)], sidx.at[slot],
                         isems[slot])
        pltpu.async_copy(dstr.at[sid, pl.ds(r * KB2, KB2)], didx.at[slot],
                         isems[slot])

    def wait_idx(r, slot):
        pltpu.make_async_copy(srcr.at[sid, pl.ds(r * KB2, KB2)],
                              sidx.at[slot], isems[slot]).wait()
        pltpu.make_async_copy(dstr.at[sid, pl.ds(r * KB2, KB2)],
                              didx.at[slot], isems[slot]).wait()

    prefetch(0, 0)
    prefetch(1, 1)

    def body(q, c):
        for rr in range(4):
            r = q * 4 + rr
            p = rr % 2
            pslot = (rr + 2) % 4
            for b in range(KB2):
                @pl.when(r >= 2)
                def _():
                    pltpu.make_async_copy(
                        rows[p][b], acc.at[didx.at[pslot, b]],
                        ssems[p][b]).wait()

            @pl.when(r + 2 < NR)
            def _():
                prefetch(r + 2, pslot)

            wait_idx(r, rr)
            for b in range(KB2):
                pltpu.async_copy(tsh.at[sidx.at[rr, b]], rows[p][b],
                                 gsems[p][b])
            for b in range(KB2):
                pltpu.make_async_copy(tsh.at[sidx.at[rr, b]], rows[p][b],
                                      gsems[p][b]).wait()
                pltpu.async_copy(rows[p][b], acc.at[didx.at[rr, b]],
                                 ssems[p][b], add=True)
        return c

    lax.fori_loop(0, NR // 4, body, 0)
    for rr in (NR - 2) % 4, (NR - 1) % 4:
        p = rr % 2
        for b in range(KB2):
            pltpu.make_async_copy(rows[p][b], acc.at[didx.at[rr, b]],
                                  ssems[p][b]).wait()
    plsc.subcore_barrier()

    @pl.when(cid == 0)
    def _():
        pltpu.sync_copy(acc.at[pl.ds(sid * RPS, RPS)],
                        out.at[pl.ds(sid * RPS, RPS), pl.ds(0, HC)])

    @pl.when(cid == 1)
    def _():
        pltpu.sync_copy(acc.at[pl.ds(sid * RPS, RPS)],
                        out.at[pl.ds(sid * RPS, RPS), pl.ds(HC, HC)])


_sc_gs = pl.kernel(
    _sc_gs_body,
    out_type=jax.ShapeDtypeStruct((NP, HH), jnp.float32),
    mesh=_MESH,
    compiler_params=_SC_PARAMS,
    scratch_types=(
        [pltpu.VMEM((4, KB2, CHUNK), jnp.int32),
         pltpu.VMEM((4, KB2, CHUNK), jnp.int32)]
        + [pltpu.VMEM((CHUNK, HC), jnp.float32) for _ in range(2 * KB2)]
        + [pltpu.VMEM_SHARED((NP, HC), jnp.float32),
           pltpu.VMEM_SHARED((NP, HC), jnp.float32)]
        + [pltpu.SemaphoreType.DMA for _ in range(4 * KB2 + 4)]
    ),
)


def _dinv_block(dega_ref, degb_ref):
    return lax.rsqrt(1.0 + dega_ref[0][:, :1] + degb_ref[0][:, :1])


def _tc_a_body(x_ref, w0_ref, b0_ref, w1_ref, b1_ref, dega_ref, degb_ref,
               out_ref):
    dinv = _dinv_block(dega_ref, degb_ref)
    h0 = jnp.dot(x_ref[...], w0_ref[...], precision=_PREC,
                 preferred_element_type=jnp.float32) + b0_ref[...]
    out_ref[...] = dinv * (jnp.dot(h0, w1_ref[...], precision=_PREC,
                                   preferred_element_type=jnp.float32)
                           + b1_ref[...])


def _tc_b_body(s_ref, ht_ref, dega_ref, degb_ref, w_ref, b_ref, out_ref):
    dinv = _dinv_block(dega_ref, degb_ref)
    t = jnp.maximum(dinv * (s_ref[...] + ht_ref[...]), 0.0)
    out_ref[...] = dinv * (jnp.dot(t, w_ref[...], precision=_PREC,
                                   preferred_element_type=jnp.float32)
                           + b_ref[...])


def _tc_c_body(s_ref, ht_ref, dega_ref, degb_ref, batch_ref, out_ref):
    dinv = _dinv_block(dega_ref, degb_ref)
    agg = dinv * (s_ref[...] + ht_ref[...])
    onehot = (batch_ref[...] ==
              lax.broadcasted_iota(jnp.int32, (BM, GG), 1)).astype(jnp.float32)
    contrib = lax.dot_general(onehot, agg, (((0,), (0,)), ((), ())),
                              precision=_PREC,
                              preferred_element_type=jnp.float32)

    @pl.when(pl.program_id(0) == 0)
    def _():
        out_ref[...] = jnp.zeros_like(out_ref)

    out_ref[...] += contrib


_rows_spec = pl.BlockSpec((BM, HH), lambda i: (i, 0))
_dega_spec = pl.BlockSpec((1, BM, DEGW), lambda i: (0, i, 0))
_degb_spec = pl.BlockSpec((1, BM, DEGW), lambda i: (1, i, 0))
_w_spec = pl.BlockSpec((HH, HH), lambda i: (0, 0))
_b_spec = pl.BlockSpec((1, HH), lambda i: (0, 0))
_rows_shape = jax.ShapeDtypeStruct((NP, HH), jnp.float32)

_tc_a = pl.pallas_call(
    _tc_a_body,
    grid=(GRID,),
    in_specs=[_rows_spec, _w_spec, _b_spec, _w_spec, _b_spec,
              _dega_spec, _degb_spec],
    out_specs=_rows_spec,
    out_shape=_rows_shape,
)

_tc_b = pl.pallas_call(
    _tc_b_body,
    grid=(GRID,),
    in_specs=[_rows_spec, _rows_spec, _dega_spec, _degb_spec,
              _w_spec, _b_spec],
    out_specs=_rows_spec,
    out_shape=_rows_shape,
)

_tc_c = pl.pallas_call(
    _tc_c_body,
    grid=(GRID,),
    in_specs=[_rows_spec, _rows_spec, _dega_spec, _degb_spec,
              pl.BlockSpec((BM, 1), lambda i: (i, 0))],
    out_specs=pl.BlockSpec((GG, HH), lambda i: (0, 0)),
    out_shape=jax.ShapeDtypeStruct((GG, HH), jnp.float32),
)


def kernel(x, edge_index, batch, W0, b0, W1, b1, W2, b2, W3, b3):
    f32 = jnp.float32
    xp = jnp.zeros((NP, HH), f32).at[:NN, :DIN].set(x)
    w0p = jnp.zeros((HH, HH), f32).at[:DIN].set(W0)
    b0r = b0.reshape(1, HH)
    b1r = b1.reshape(1, HH)
    b2r = b2.reshape(1, HH)
    b3r = b3.reshape(1, HH)

    epad = EP - EE
    srcp = jnp.concatenate(
        [edge_index[0], jnp.full((epad,), NN, jnp.int32)]).reshape(
            NS, NCH, CHUNK)
    dstp = jnp.concatenate(
        [edge_index[1], jnp.full((epad,), NN, jnp.int32)]).reshape(
            NS, NCH, CHUNK)
    batchp = jnp.concatenate(
        [batch, jnp.full((NP - NN,), GG, jnp.int32)]).reshape(NP, 1)

    zeros_c = jnp.zeros((RPS, HC), f32)
    zeros_d = jnp.zeros((RPS, DEGW), f32)
    ones_d = jnp.ones((CHUNK, DEGW), f32)

    deg2 = _sc_deg(dstp, zeros_d, ones_d)
    ht1 = _tc_a(xp, w0p, b0r, W1, b1r, deg2, deg2)
    s1 = _sc_gs(ht1, srcp, dstp, zeros_c)
    ht2 = _tc_b(s1, ht1, deg2, deg2, W2, b2r)
    s2 = _sc_gs(ht2, srcp, dstp, zeros_c)
    ht3 = _tc_b(s2, ht2, deg2, deg2, W3, b3r)
    s3 = _sc_gs(ht3, srcp, dstp, zeros_c)
    pooled = _tc_c(s3, ht3, deg2, deg2, batchp)
    return pooled


# speedup vs baseline: 16.9881x; 1.0179x over previous
import jax
import jax.numpy as jnp
from jax import lax
from jax.experimental import pallas as pl
from jax.experimental.pallas import tpu as pltpu
from jax.experimental.pallas import tpu_sc as plsc

NN = 10000
EE = 320000
DIN = 70
HH = 128
HC = HH // 2
GG = 256

NC = 2
NS = 16
NP = 10240
RPS = NP // NS
EPW = 20480
EP = NS * EPW
CHUNK = 128
NCH = EPW // CHUNK
KB2 = 2
NR = NCH // KB2
DEGW = 16

BM = 1280
GRID = NP // BM

_MESH = plsc.VectorSubcoreMesh(core_axis_name="c", subcore_axis_name="s",
                               num_cores=NC, num_subcores=NS)
_SC_PARAMS = pltpu.CompilerParams(use_tc_tiling_on_sc=False)
_PREC = lax.Precision.HIGHEST


def _sc_deg_body(dstr, zeros_d, ones_d, out, dst_v, ones_v, acc):
    cid = lax.axis_index("c")
    sid = lax.axis_index("s")
    pltpu.sync_copy(zeros_d, acc.at[pl.ds(sid * RPS, RPS)])
    pltpu.sync_copy(ones_d, ones_v)
    pltpu.sync_copy(dstr.at[sid], dst_v)
    plsc.subcore_barrier()

    def body(j, c):
        pltpu.sync_copy(ones_v, acc.at[dst_v.at[j]], add=True)
        return c

    lax.fori_loop(cid * (NCH // 2), (cid + 1) * (NCH // 2), body, 0)
    plsc.subcore_barrier()

    @pl.when(cid == 0)
    def _():
        pltpu.sync_copy(acc.at[pl.ds(sid * RPS, RPS)],
                        out.at[pl.ds(sid * RPS, RPS), pl.ds(0, DEGW)])

    @pl.when(cid == 1)
    def _():
        pltpu.sync_copy(acc.at[pl.ds(sid * RPS, RPS)],
                        out.at[pl.ds(sid * RPS, RPS), pl.ds(DEGW, DEGW)])


_sc_deg = pl.kernel(
    _sc_deg_body,
    out_type=jax.ShapeDtypeStruct((NP, HH), jnp.float32),
    mesh=_MESH,
    compiler_params=_SC_PARAMS,
    scratch_types=[
        pltpu.VMEM((NCH, CHUNK), jnp.int32),
        pltpu.VMEM((CHUNK, DEGW), jnp.float32),
        pltpu.VMEM_SHARED((NP, DEGW), jnp.float32),
    ],
)


def _sc_gs_body(ht, srcr, dstr, zeros_c, out, *scratch):
    sidx, didx = scratch[0], scratch[1]
    rows = (scratch[2:4], scratch[4:6])
    tsh = scratch[6]
    acc = scratch[7]
    gsems = (scratch[8:10], scratch[10:12])
    ssems = (scratch[12:14], scratch[14:16])
    isems = scratch[16:20]
    cid = lax.axis_index("c")
    sid = lax.axis_index("s")

    pltpu.sync_copy(zeros_c, acc.at[pl.ds(sid * RPS, RPS)])

    @pl.when(cid == 0)
    def _():
        pltpu.sync_copy(ht.at[pl.ds(sid * RPS, RPS), pl.ds(0, HC)],
                        tsh.at[pl.ds(sid * RPS, RPS)])

    @pl.when(cid == 1)
    def _():
        pltpu.sync_copy(ht.at[pl.ds(sid * RPS, RPS), pl.ds(HC, HC)],
                        tsh.at[pl.ds(sid * RPS, RPS)])

    plsc.subcore_barrier()

    def prefetch(r, slot):
        pltpu.async_copy(srcr.at[sid, pl.ds(r * KB2, KB2)], sidx.at[slot],
                         isems[slot])
        pltpu.async_copy(dstr.at[sid, pl.ds(r * KB2, KB2)], didx.at[slot],
                         isems[slot])

    def wait_idx(r, slot):
        pltpu.make_async_copy(srcr.at[sid, pl.ds(r * KB2, KB2)],
                              sidx.at[slot], isems[slot]).wait()
        pltpu.make_async_copy(dstr.at[sid, pl.ds(r * KB2, KB2)],
                              didx.at[slot], isems[slot]).wait()

    prefetch(0, 0)
    prefetch(1, 1)

    def body(q, c):
        for rr in range(4):
            r = q * 4 + rr
            p = rr % 2
            pslot = (rr + 2) % 4
            for b in range(KB2):
                @pl.when(r >= 2)
                def _():
                    pltpu.make_async_copy(
                        rows[p][b], acc.at[didx.at[pslot, b]],
                        ssems[p][b]).wait()

            @pl.when(r + 2 < NR)
            def _():
                prefetch(r + 2, pslot)

            wait_idx(r, rr)
            for b in range(KB2):
                pltpu.async_copy(tsh.at[sidx.at[rr, b]], rows[p][b],
                                 gsems[p][b])
            for b in range(KB2):
                pltpu.make_async_copy(tsh.at[sidx.at[rr, b]], rows[p][b],
                                      gsems[p][b]).wait()
                pltpu.async_copy(rows[p][b], acc.at[didx.at[rr, b]],
                                 ssems[p][b], add=True)
        return c

    lax.fori_loop(0, NR // 4, body, 0)
    for rr in (NR - 2) % 4, (NR - 1) % 4:
        p = rr % 2
        for b in range(KB2):
            pltpu.make_async_copy(rows[p][b], acc.at[didx.at[rr, b]],
                                  ssems[p][b]).wait()
    plsc.subcore_barrier()

    @pl.when(cid == 0)
    def _():
        pltpu.sync_copy(acc.at[pl.ds(sid * RPS, RPS)],
                        out.at[pl.ds(sid * RPS, RPS), pl.ds(0, HC)])

    @pl.when(cid == 1)
    def _():
        pltpu.sync_copy(acc.at[pl.ds(sid * RPS, RPS)],
                        out.at[pl.ds(sid * RPS, RPS), pl.ds(HC, HC)])


_sc_gs = pl.kernel(
    _sc_gs_body,
    out_type=jax.ShapeDtypeStruct((NP, HH), jnp.float32),
    mesh=_MESH,
    compiler_params=_SC_PARAMS,
    scratch_types=(
        [pltpu.VMEM((4, KB2, CHUNK), jnp.int32),
         pltpu.VMEM((4, KB2, CHUNK), jnp.int32)]
        + [pltpu.VMEM((CHUNK, HC), jnp.float32) for _ in range(2 * KB2)]
        + [pltpu.VMEM_SHARED((NP, HC), jnp.float32),
           pltpu.VMEM_SHARED((NP, HC), jnp.float32)]
        + [pltpu.SemaphoreType.DMA for _ in range(4 * KB2 + 4)]
    ),
)


def _dinv_block(deg_ref):
    return lax.rsqrt(1.0 + deg_ref[:, 0:1] + deg_ref[:, DEGW:DEGW + 1])


def _tc_m_body(x_ref, w0_ref, b0_ref, w1_ref, b1_ref, out_ref):
    h0 = jnp.dot(x_ref[...], w0_ref[...], precision=_PREC,
                 preferred_element_type=jnp.float32) + b0_ref[...]
    out_ref[...] = (jnp.dot(h0, w1_ref[...], precision=_PREC,
                            preferred_element_type=jnp.float32)
                    + b1_ref[...])


def _tc_scale_body(h_ref, deg_ref, out_ref):
    out_ref[...] = _dinv_block(deg_ref) * h_ref[...]


def _tc_b_body(s_ref, ht_ref, deg_ref, w_ref, b_ref, out_ref):
    dinv = _dinv_block(deg_ref)
    t = jnp.maximum(dinv * (s_ref[...] + ht_ref[...]), 0.0)
    out_ref[...] = dinv * (jnp.dot(t, w_ref[...], precision=_PREC,
                                   preferred_element_type=jnp.float32)
                           + b_ref[...])


def _tc_c_body(s_ref, ht_ref, deg_ref, batch_ref, out_ref):
    dinv = _dinv_block(deg_ref)
    agg = jnp.where(batch_ref[...] < GG,
                    dinv * (s_ref[...] + ht_ref[...]), 0.0)
    onehot = (batch_ref[...] ==
              lax.broadcasted_iota(jnp.int32, (BM, GG), 1)).astype(jnp.float32)
    contrib = lax.dot_general(onehot, agg, (((0,), (0,)), ((), ())),
                              precision=_PREC,
                              preferred_element_type=jnp.float32)

    @pl.when(pl.program_id(0) == 0)
    def _():
        out_ref[...] = jnp.zeros_like(out_ref)

    out_ref[...] += contrib


_rows_spec = pl.BlockSpec((BM, HH), lambda i: (i, 0))
_x_spec = pl.BlockSpec((BM, DIN), lambda i: (i, 0))
_w0_spec = pl.BlockSpec((DIN, HH), lambda i: (0, 0))
_w_spec = pl.BlockSpec((HH, HH), lambda i: (0, 0))
_b_spec = pl.BlockSpec((1, HH), lambda i: (0, 0))
_batch_spec = pl.BlockSpec((BM, 1), lambda i: (i, 0))
_rows_shape = jax.ShapeDtypeStruct((NP, HH), jnp.float32)

_tc_m = pl.pallas_call(
    _tc_m_body,
    grid=(GRID,),
    in_specs=[_x_spec, _w0_spec, _b_spec, _w_spec, _b_spec],
    out_specs=_rows_spec,
    out_shape=_rows_shape,
)

_tc_scale = pl.pallas_call(
    _tc_scale_body,
    grid=(GRID,),
    in_specs=[_rows_spec, _rows_spec],
    out_specs=_rows_spec,
    out_shape=_rows_shape,
)

_tc_b = pl.pallas_call(
    _tc_b_body,
    grid=(GRID,),
    in_specs=[_rows_spec, _rows_spec, _rows_spec, _w_spec, _b_spec],
    out_specs=_rows_spec,
    out_shape=_rows_shape,
)

_tc_c = pl.pallas_call(
    _tc_c_body,
    grid=(GRID,),
    in_specs=[_rows_spec, _rows_spec, _rows_spec, _batch_spec],
    out_specs=pl.BlockSpec((GG, HH), lambda i: (0, 0)),
    out_shape=jax.ShapeDtypeStruct((GG, HH), jnp.float32),
)


def kernel(x, edge_index, batch, W0, b0, W1, b1, W2, b2, W3, b3):
    f32 = jnp.float32
    b0r = b0.reshape(1, HH)
    b1r = b1.reshape(1, HH)
    b2r = b2.reshape(1, HH)
    b3r = b3.reshape(1, HH)

    epad = EP - EE
    srcp = jnp.concatenate(
        [edge_index[0], jnp.full((epad,), NN, jnp.int32)]).reshape(
            NS, NCH, CHUNK)
    dstp = jnp.concatenate(
        [edge_index[1], jnp.full((epad,), NN, jnp.int32)]).reshape(
            NS, NCH, CHUNK)
    batchp = jnp.concatenate(
        [batch, jnp.full((NP - NN,), GG, jnp.int32)]).reshape(NP, 1)

    zeros_c = jnp.zeros((RPS, HC), f32)
    zeros_d = jnp.zeros((RPS, DEGW), f32)
    ones_d = jnp.ones((CHUNK, DEGW), f32)

    deg2 = _sc_deg(dstp, zeros_d, ones_d)
    hh1 = _tc_m(x, W0, b0r, W1, b1r)
    ht1 = _tc_scale(hh1, deg2)
    s1 = _sc_gs(ht1, srcp, dstp, zeros_c)
    ht2 = _tc_b(s1, ht1, deg2, W2, b2r)
    s2 = _sc_gs(ht2, srcp, dstp, zeros_c)
    ht3 = _tc_b(s2, ht2, deg2, W3, b3r)
    s3 = _sc_gs(ht3, srcp, dstp, zeros_c)
    pooled = _tc_c(s3, ht3, deg2, batchp)
    return pooled


# speedup vs baseline: 17.2117x; 1.0132x over previous
import jax
import jax.numpy as jnp
from jax import lax
from jax.experimental import pallas as pl
from jax.experimental.pallas import tpu as pltpu
from jax.experimental.pallas import tpu_sc as plsc

NN = 10000
EE = 320000
DIN = 70
HH = 128
HC = HH // 2
GG = 256

NC = 2
NS = 16
NP = 10240
RPS = NP // NS
EPW = 20480
EP = NS * EPW
CHUNK = 128
NCH = EPW // CHUNK
KB2 = 2
NR = NCH // KB2
DEGW = 16

BM = 1280
GRID = NP // BM

_MESH = plsc.VectorSubcoreMesh(core_axis_name="c", subcore_axis_name="s",
                               num_cores=NC, num_subcores=NS)
_SC_PARAMS = pltpu.CompilerParams(use_tc_tiling_on_sc=False)
_PREC = lax.Precision.DEFAULT


def _sc_deg_body(dstr, zeros_d, ones_d, out, dst_v, ones_v, acc):
    cid = lax.axis_index("c")
    sid = lax.axis_index("s")
    pltpu.sync_copy(zeros_d, acc.at[pl.ds(sid * RPS, RPS)])
    pltpu.sync_copy(ones_d, ones_v)
    pltpu.sync_copy(dstr.at[sid], dst_v)
    plsc.subcore_barrier()

    def body(j, c):
        pltpu.sync_copy(ones_v, acc.at[dst_v.at[j]], add=True)
        return c

    lax.fori_loop(cid * (NCH // 2), (cid + 1) * (NCH // 2), body, 0)
    plsc.subcore_barrier()

    @pl.when(cid == 0)
    def _():
        pltpu.sync_copy(acc.at[pl.ds(sid * RPS, RPS)],
                        out.at[pl.ds(sid * RPS, RPS), pl.ds(0, DEGW)])

    @pl.when(cid == 1)
    def _():
        pltpu.sync_copy(acc.at[pl.ds(sid * RPS, RPS)],
                        out.at[pl.ds(sid * RPS, RPS), pl.ds(DEGW, DEGW)])


_sc_deg = pl.kernel(
    _sc_deg_body,
    out_type=jax.ShapeDtypeStruct((NP, HH), jnp.float32),
    mesh=_MESH,
    compiler_params=_SC_PARAMS,
    scratch_types=[
        pltpu.VMEM((NCH, CHUNK), jnp.int32),
        pltpu.VMEM((CHUNK, DEGW), jnp.float32),
        pltpu.VMEM_SHARED((NP, DEGW), jnp.float32),
    ],
)


def _sc_gs_body(ht, srcr, dstr, zeros_c, out, *scratch):
    sidx, didx = scratch[0], scratch[1]
    rows = (scratch[2:4], scratch[4:6])
    tsh = scratch[6]
    acc = scratch[7]
    gsems = (scratch[8:10], scratch[10:12])
    ssems = (scratch[12:14], scratch[14:16])
    isems = scratch[16:20]
    cid = lax.axis_index("c")
    sid = lax.axis_index("s")

    pltpu.sync_copy(zeros_c, acc.at[pl.ds(sid * RPS, RPS)])

    @pl.when(cid == 0)
    def _():
        pltpu.sync_copy(ht.at[pl.ds(sid * RPS, RPS), pl.ds(0, HC)],
                        tsh.at[pl.ds(sid * RPS, RPS)])

    @pl.when(cid == 1)
    def _():
        pltpu.sync_copy(ht.at[pl.ds(sid * RPS, RPS), pl.ds(HC, HC)],
                        tsh.at[pl.ds(sid * RPS, RPS)])

    plsc.subcore_barrier()

    def prefetch(r, slot):
        pltpu.async_copy(srcr.at[sid, pl.ds(r * KB2, KB2)], sidx.at[slot],
                         isems[slot])
        pltpu.async_copy(dstr.at[sid, pl.ds(r * KB2, KB2)], didx.at[slot],
                         isems[slot])

    def wait_idx(r, slot):
        pltpu.make_async_copy(srcr.at[sid, pl.ds(r * KB2, KB2)],
                              sidx.at[slot], isems[slot]).wait()
        pltpu.make_async_copy(dstr.at[sid, pl.ds(r * KB2, KB2)],
                              didx.at[slot], isems[slot]).wait()

    prefetch(0, 0)
    prefetch(1, 1)

    def body(q, c):
        for rr in range(4):
            r = q * 4 + rr
            p = rr % 2
            pslot = (rr + 2) % 4
            for b in range(KB2):
                @pl.when(r >= 2)
                def _():
                    pltpu.make_async_copy(
                        rows[p][b], acc.at[didx.at[pslot, b]],
                        ssems[p][b]).wait()

            @pl.when(r + 2 < NR)
            def _():
                prefetch(r + 2, pslot)

            wait_idx(r, rr)
            for b in range(KB2):
                pltpu.async_copy(tsh.at[sidx.at[rr, b]], rows[p][b],
                                 gsems[p][b])
            for b in range(KB2):
                pltpu.make_async_copy(tsh.at[sidx.at[rr, b]], rows[p][b],
                                      gsems[p][b]).wait()
                pltpu.async_copy(rows[p][b], acc.at[didx.at[rr, b]],
                                 ssems[p][b], add=True)
        return c

    lax.fori_loop(0, NR // 4, body, 0)
    for rr in (NR - 2) % 4, (NR - 1) % 4:
        p = rr % 2
        for b in range(KB2):
            pltpu.make_async_copy(rows[p][b], acc.at[didx.at[rr, b]],
                                  ssems[p][b]).wait()
    plsc.subcore_barrier()

    @pl.when(cid == 0)
    def _():
        pltpu.sync_copy(acc.at[pl.ds(sid * RPS, RPS)],
                        out.at[pl.ds(sid * RPS, RPS), pl.ds(0, HC)])

    @pl.when(cid == 1)
    def _():
        pltpu.sync_copy(acc.at[pl.ds(sid * RPS, RPS)],
                        out.at[pl.ds(sid * RPS, RPS), pl.ds(HC, HC)])


_sc_gs = pl.kernel(
    _sc_gs_body,
    out_type=jax.ShapeDtypeStruct((NP, HH), jnp.float32),
    mesh=_MESH,
    compiler_params=_SC_PARAMS,
    scratch_types=(
        [pltpu.VMEM((4, KB2, CHUNK), jnp.int32),
         pltpu.VMEM((4, KB2, CHUNK), jnp.int32)]
        + [pltpu.VMEM((CHUNK, HC), jnp.float32) for _ in range(2 * KB2)]
        + [pltpu.VMEM_SHARED((NP, HC), jnp.float32),
           pltpu.VMEM_SHARED((NP, HC), jnp.float32)]
        + [pltpu.SemaphoreType.DMA for _ in range(4 * KB2 + 4)]
    ),
)


def _dinv_block(deg_ref):
    return lax.rsqrt(1.0 + deg_ref[:, 0:1] + deg_ref[:, DEGW:DEGW + 1])


def _tc_m_body(x_ref, w0_ref, b0_ref, w1_ref, b1_ref, out_ref):
    h0 = jnp.dot(x_ref[...], w0_ref[...], precision=_PREC,
                 preferred_element_type=jnp.float32) + b0_ref[...]
    out_ref[...] = (jnp.dot(h0, w1_ref[...], precision=_PREC,
                            preferred_element_type=jnp.float32)
                    + b1_ref[...])


def _tc_scale_body(h_ref, deg_ref, out_ref):
    out_ref[...] = _dinv_block(deg_ref) * h_ref[...]


def _tc_b_body(s_ref, ht_ref, deg_ref, w_ref, b_ref, out_ref):
    dinv = _dinv_block(deg_ref)
    t = jnp.maximum(dinv * (s_ref[...] + ht_ref[...]), 0.0)
    out_ref[...] = dinv * (jnp.dot(t, w_ref[...], precision=_PREC,
                                   preferred_element_type=jnp.float32)
                           + b_ref[...])


def _tc_c_body(s_ref, ht_ref, deg_ref, batch_ref, out_ref):
    dinv = _dinv_block(deg_ref)
    agg = jnp.where(batch_ref[...] < GG,
                    dinv * (s_ref[...] + ht_ref[...]), 0.0)
    onehot = (batch_ref[...] ==
              lax.broadcasted_iota(jnp.int32, (BM, GG), 1)).astype(jnp.float32)
    contrib = lax.dot_general(onehot, agg, (((0,), (0,)), ((), ())),
                              precision=_PREC,
                              preferred_element_type=jnp.float32)

    @pl.when(pl.program_id(0) == 0)
    def _():
        out_ref[...] = jnp.zeros_like(out_ref)

    out_ref[...] += contrib


_rows_spec = pl.BlockSpec((BM, HH), lambda i: (i, 0))
_x_spec = pl.BlockSpec((BM, DIN), lambda i: (i, 0))
_w0_spec = pl.BlockSpec((DIN, HH), lambda i: (0, 0))
_w_spec = pl.BlockSpec((HH, HH), lambda i: (0, 0))
_b_spec = pl.BlockSpec((1, HH), lambda i: (0, 0))
_batch_spec = pl.BlockSpec((BM, 1), lambda i: (i, 0))
_rows_shape = jax.ShapeDtypeStruct((NP, HH), jnp.float32)

_tc_m = pl.pallas_call(
    _tc_m_body,
    grid=(GRID,),
    in_specs=[_x_spec, _w0_spec, _b_spec, _w_spec, _b_spec],
    out_specs=_rows_spec,
    out_shape=_rows_shape,
)

_tc_scale = pl.pallas_call(
    _tc_scale_body,
    grid=(GRID,),
    in_specs=[_rows_spec, _rows_spec],
    out_specs=_rows_spec,
    out_shape=_rows_shape,
)

_tc_b = pl.pallas_call(
    _tc_b_body,
    grid=(GRID,),
    in_specs=[_rows_spec, _rows_spec, _rows_spec, _w_spec, _b_spec],
    out_specs=_rows_spec,
    out_shape=_rows_shape,
)

_tc_c = pl.pallas_call(
    _tc_c_body,
    grid=(GRID,),
    in_specs=[_rows_spec, _rows_spec, _rows_spec, _batch_spec],
    out_specs=pl.BlockSpec((GG, HH), lambda i: (0, 0)),
    out_shape=jax.ShapeDtypeStruct((GG, HH), jnp.float32),
)


def kernel(x, edge_index, batch, W0, b0, W1, b1, W2, b2, W3, b3):
    f32 = jnp.float32
    b0r = b0.reshape(1, HH)
    b1r = b1.reshape(1, HH)
    b2r = b2.reshape(1, HH)
    b3r = b3.reshape(1, HH)

    epad = EP - EE
    srcp = jnp.concatenate(
        [edge_index[0], jnp.full((epad,), NN, jnp.int32)]).reshape(
            NS, NCH, CHUNK)
    dstp = jnp.concatenate(
        [edge_index[1], jnp.full((epad,), NN, jnp.int32)]).reshape(
            NS, NCH, CHUNK)
    batchp = jnp.concatenate(
        [batch, jnp.full((NP - NN,), GG, jnp.int32)]).reshape(NP, 1)

    zeros_c = jnp.zeros((RPS, HC), f32)
    zeros_d = jnp.zeros((RPS, DEGW), f32)
    ones_d = jnp.ones((CHUNK, DEGW), f32)

    deg2 = _sc_deg(dstp, zeros_d, ones_d)
    hh1 = _tc_m(x, W0, b0r, W1, b1r)
    ht1 = _tc_scale(hh1, deg2)
    s1 = _sc_gs(ht1, srcp, dstp, zeros_c)
    ht2 = _tc_b(s1, ht1, deg2, W2, b2r)
    s2 = _sc_gs(ht2, srcp, dstp, zeros_c)
    ht3 = _tc_b(s2, ht2, deg2, W3, b3r)
    s3 = _sc_gs(ht3, srcp, dstp, zeros_c)
    pooled = _tc_c(s3, ht3, deg2, batchp)
    return pooled


# speedup vs baseline: 19.8824x; 1.1552x over previous
import jax
import jax.numpy as jnp
from jax import lax
from jax.experimental import pallas as pl
from jax.experimental.pallas import tpu as pltpu
from jax.experimental.pallas import tpu_sc as plsc

NN = 10000
EE = 320000
DIN = 70
HH = 128
HC = HH // 2
GG = 256

NC = 2
NS = 16
NP = 10240
RPS = NP // NS
EPW = 20480
EP = NS * EPW
CHUNK = 128
NCH = EPW // CHUNK
KB2 = 2
NR = NCH // KB2
DEGW = 16

BM = 1280
GRID = NP // BM

_MESH = plsc.VectorSubcoreMesh(core_axis_name="c", subcore_axis_name="s",
                               num_cores=NC, num_subcores=NS)
_SC_PARAMS = pltpu.CompilerParams(use_tc_tiling_on_sc=False)
_PREC = lax.Precision.DEFAULT


def _sc_deg_body(dstr, zeros_d, ones_d, out, dst_v, ones_v, acc):
    cid = lax.axis_index("c")
    sid = lax.axis_index("s")
    pltpu.sync_copy(zeros_d, acc.at[pl.ds(sid * RPS, RPS)])
    pltpu.sync_copy(ones_d, ones_v)
    pltpu.sync_copy(dstr.at[sid], dst_v)
    plsc.subcore_barrier()

    def body(j, c):
        pltpu.sync_copy(ones_v, acc.at[dst_v.at[j]], add=True)
        return c

    lax.fori_loop(cid * (NR // 2), (cid + 1) * (NR // 2), body, 0)
    plsc.subcore_barrier()

    @pl.when(cid == 0)
    def _():
        pltpu.sync_copy(acc.at[pl.ds(sid * RPS, RPS)],
                        out.at[pl.ds(sid * RPS, RPS), pl.ds(0, DEGW)])

    @pl.when(cid == 1)
    def _():
        pltpu.sync_copy(acc.at[pl.ds(sid * RPS, RPS)],
                        out.at[pl.ds(sid * RPS, RPS), pl.ds(DEGW, DEGW)])


_sc_deg = pl.kernel(
    _sc_deg_body,
    out_type=jax.ShapeDtypeStruct((NP, HH), jnp.float32),
    mesh=_MESH,
    compiler_params=_SC_PARAMS,
    scratch_types=[
        pltpu.VMEM((NR, KB2 * CHUNK), jnp.int32),
        pltpu.VMEM((KB2 * CHUNK, DEGW), jnp.float32),
        pltpu.VMEM_SHARED((NP, DEGW), jnp.float32),
    ],
)


def _sc_gs_body(ht, srcr, dstr, zeros_c, out, *scratch):
    sidx, didx = scratch[0], scratch[1]
    rows = scratch[2:4]
    tsh = scratch[4]
    acc = scratch[5]
    gsems = scratch[6:8]
    ssems = scratch[8:10]
    isems = scratch[10:14]
    cid = lax.axis_index("c")
    sid = lax.axis_index("s")

    pltpu.sync_copy(zeros_c, acc.at[pl.ds(sid * RPS, RPS)])

    @pl.when(cid == 0)
    def _():
        pltpu.sync_copy(ht.at[pl.ds(sid * RPS, RPS), pl.ds(0, HC)],
                        tsh.at[pl.ds(sid * RPS, RPS)])

    @pl.when(cid == 1)
    def _():
        pltpu.sync_copy(ht.at[pl.ds(sid * RPS, RPS), pl.ds(HC, HC)],
                        tsh.at[pl.ds(sid * RPS, RPS)])

    plsc.subcore_barrier()

    def prefetch(r, slot):
        pltpu.async_copy(srcr.at[sid, r], sidx.at[slot], isems[slot])
        pltpu.async_copy(dstr.at[sid, r], didx.at[slot], isems[slot])

    def wait_idx(r, slot):
        pltpu.make_async_copy(srcr.at[sid, r], sidx.at[slot],
                              isems[slot]).wait()
        pltpu.make_async_copy(dstr.at[sid, r], didx.at[slot],
                              isems[slot]).wait()

    prefetch(0, 0)
    prefetch(1, 1)

    def body(q, c):
        for rr in range(4):
            r = q * 4 + rr
            p = rr % 2
            pslot = (rr + 2) % 4
            @pl.when(r >= 2)
            def _():
                pltpu.make_async_copy(rows[p], acc.at[didx.at[pslot]],
                                      ssems[p]).wait()

            @pl.when(r + 2 < NR)
            def _():
                prefetch(r + 2, pslot)

            wait_idx(r, rr)
            pltpu.async_copy(tsh.at[sidx.at[rr]], rows[p], gsems[p])
            pltpu.make_async_copy(tsh.at[sidx.at[rr]], rows[p],
                                  gsems[p]).wait()
            pltpu.async_copy(rows[p], acc.at[didx.at[rr]], ssems[p],
                             add=True)
        return c

    lax.fori_loop(0, NR // 4, body, 0)
    for rr in (NR - 2) % 4, (NR - 1) % 4:
        p = rr % 2
        pltpu.make_async_copy(rows[p], acc.at[didx.at[rr]],
                              ssems[p]).wait()
    plsc.subcore_barrier()

    @pl.when(cid == 0)
    def _():
        pltpu.sync_copy(acc.at[pl.ds(sid * RPS, RPS)],
                        out.at[pl.ds(sid * RPS, RPS), pl.ds(0, HC)])

    @pl.when(cid == 1)
    def _():
        pltpu.sync_copy(acc.at[pl.ds(sid * RPS, RPS)],
                        out.at[pl.ds(sid * RPS, RPS), pl.ds(HC, HC)])


_sc_gs = pl.kernel(
    _sc_gs_body,
    out_type=jax.ShapeDtypeStruct((NP, HH), jnp.float32),
    mesh=_MESH,
    compiler_params=_SC_PARAMS,
    scratch_types=(
        [pltpu.VMEM((4, KB2 * CHUNK), jnp.int32),
         pltpu.VMEM((4, KB2 * CHUNK), jnp.int32)]
        + [pltpu.VMEM((KB2 * CHUNK, HC), jnp.float32) for _ in range(2)]
        + [pltpu.VMEM_SHARED((NP, HC), jnp.float32),
           pltpu.VMEM_SHARED((NP, HC), jnp.float32)]
        + [pltpu.SemaphoreType.DMA for _ in range(8)]
    ),
)


def _dinv_block(deg_ref):
    return lax.rsqrt(1.0 + deg_ref[:, 0:1] + deg_ref[:, DEGW:DEGW + 1])


def _tc_m_body(x_ref, w0_ref, b0_ref, w1_ref, b1_ref, out_ref):
    h0 = jnp.dot(x_ref[...], w0_ref[...], precision=_PREC,
                 preferred_element_type=jnp.float32) + b0_ref[...]
    out_ref[...] = (jnp.dot(h0, w1_ref[...], precision=_PREC,
                            preferred_element_type=jnp.float32)
                    + b1_ref[...])


def _tc_scale_body(h_ref, deg_ref, out_ref):
    out_ref[...] = _dinv_block(deg_ref) * h_ref[...]


def _tc_b_body(s_ref, ht_ref, deg_ref, w_ref, b_ref, out_ref):
    dinv = _dinv_block(deg_ref)
    t = jnp.maximum(dinv * (s_ref[...] + ht_ref[...]), 0.0)
    out_ref[...] = dinv * (jnp.dot(t, w_ref[...], precision=_PREC,
                                   preferred_element_type=jnp.float32)
                           + b_ref[...])


def _tc_c_body(s_ref, ht_ref, deg_ref, batch_ref, out_ref):
    dinv = _dinv_block(deg_ref)
    agg = jnp.where(batch_ref[...] < GG,
                    dinv * (s_ref[...] + ht_ref[...]), 0.0)
    onehot = (batch_ref[...] ==
              lax.broadcasted_iota(jnp.int32, (BM, GG), 1)).astype(jnp.float32)
    contrib = lax.dot_general(onehot, agg, (((0,), (0,)), ((), ())),
                              precision=_PREC,
                              preferred_element_type=jnp.float32)

    @pl.when(pl.program_id(0) == 0)
    def _():
        out_ref[...] = jnp.zeros_like(out_ref)

    out_ref[...] += contrib


_rows_spec = pl.BlockSpec((BM, HH), lambda i: (i, 0))
_x_spec = pl.BlockSpec((BM, DIN), lambda i: (i, 0))
_w0_spec = pl.BlockSpec((DIN, HH), lambda i: (0, 0))
_w_spec = pl.BlockSpec((HH, HH), lambda i: (0, 0))
_b_spec = pl.BlockSpec((1, HH), lambda i: (0, 0))
_batch_spec = pl.BlockSpec((BM, 1), lambda i: (i, 0))
_rows_shape = jax.ShapeDtypeStruct((NP, HH), jnp.float32)

_tc_m = pl.pallas_call(
    _tc_m_body,
    grid=(GRID,),
    in_specs=[_x_spec, _w0_spec, _b_spec, _w_spec, _b_spec],
    out_specs=_rows_spec,
    out_shape=_rows_shape,
)

_tc_scale = pl.pallas_call(
    _tc_scale_body,
    grid=(GRID,),
    in_specs=[_rows_spec, _rows_spec],
    out_specs=_rows_spec,
    out_shape=_rows_shape,
)

_tc_b = pl.pallas_call(
    _tc_b_body,
    grid=(GRID,),
    in_specs=[_rows_spec, _rows_spec, _rows_spec, _w_spec, _b_spec],
    out_specs=_rows_spec,
    out_shape=_rows_shape,
)

_tc_c = pl.pallas_call(
    _tc_c_body,
    grid=(GRID,),
    in_specs=[_rows_spec, _rows_spec, _rows_spec, _batch_spec],
    out_specs=pl.BlockSpec((GG, HH), lambda i: (0, 0)),
    out_shape=jax.ShapeDtypeStruct((GG, HH), jnp.float32),
)


def kernel(x, edge_index, batch, W0, b0, W1, b1, W2, b2, W3, b3):
    f32 = jnp.float32
    b0r = b0.reshape(1, HH)
    b1r = b1.reshape(1, HH)
    b2r = b2.reshape(1, HH)
    b3r = b3.reshape(1, HH)

    epad = EP - EE
    srcp = jnp.concatenate(
        [edge_index[0], jnp.full((epad,), NN, jnp.int32)]).reshape(
            NS, NR, KB2 * CHUNK)
    dstp = jnp.concatenate(
        [edge_index[1], jnp.full((epad,), NN, jnp.int32)]).reshape(
            NS, NR, KB2 * CHUNK)
    batchp = jnp.concatenate(
        [batch, jnp.full((NP - NN,), GG, jnp.int32)]).reshape(NP, 1)

    zeros_c = jnp.zeros((RPS, HC), f32)
    zeros_d = jnp.zeros((RPS, DEGW), f32)
    ones_d = jnp.ones((KB2 * CHUNK, DEGW), f32)

    deg2 = _sc_deg(dstp, zeros_d, ones_d)
    hh1 = _tc_m(x, W0, b0r, W1, b1r)
    ht1 = _tc_scale(hh1, deg2)
    s1 = _sc_gs(ht1, srcp, dstp, zeros_c)
    ht2 = _tc_b(s1, ht1, deg2, W2, b2r)
    s2 = _sc_gs(ht2, srcp, dstp, zeros_c)
    ht3 = _tc_b(s2, ht2, deg2, W3, b3r)
    s3 = _sc_gs(ht3, srcp, dstp, zeros_c)
    pooled = _tc_c(s3, ht3, deg2, batchp)
    return pooled


# speedup vs baseline: 20.3641x; 1.0242x over previous
import jax
import jax.numpy as jnp
from jax import lax
from jax.experimental import pallas as pl
from jax.experimental.pallas import tpu as pltpu
from jax.experimental.pallas import tpu_sc as plsc

NN = 10000
EE = 320000
DIN = 70
HH = 128
HC = HH // 2
GG = 256

NC = 2
NS = 16
NP = 10240
RPS = NP // NS
EPW = 20480
EP = NS * EPW
CHUNK = 128
NCH = EPW // CHUNK
KB2 = 2
NR = NCH // KB2
DEGW = 16

BM = 1280
GRID = NP // BM

_MESH = plsc.VectorSubcoreMesh(core_axis_name="c", subcore_axis_name="s",
                               num_cores=NC, num_subcores=NS)
_SC_PARAMS = pltpu.CompilerParams(use_tc_tiling_on_sc=False)
_PREC = lax.Precision.DEFAULT


def _sc_deg_body(eidx, zeros_d, ones_d, out, dst_v, ones_v, acc):
    cid = lax.axis_index("c")
    sid = lax.axis_index("s")
    pltpu.sync_copy(zeros_d, acc.at[pl.ds(sid * RPS, RPS)])
    pltpu.sync_copy(ones_d, ones_v)
    pltpu.sync_copy(eidx.at[1, sid], dst_v)
    plsc.subcore_barrier()

    def body(j, c):
        pltpu.sync_copy(ones_v, acc.at[dst_v.at[j]], add=True)
        return c

    lax.fori_loop(cid * (NR // 2), (cid + 1) * (NR // 2), body, 0)
    plsc.subcore_barrier()

    @pl.when(cid == 0)
    def _():
        pltpu.sync_copy(acc.at[pl.ds(sid * RPS, RPS)],
                        out.at[pl.ds(sid * RPS, RPS), pl.ds(0, DEGW)])

    @pl.when(cid == 1)
    def _():
        pltpu.sync_copy(acc.at[pl.ds(sid * RPS, RPS)],
                        out.at[pl.ds(sid * RPS, RPS), pl.ds(DEGW, DEGW)])


_sc_deg = pl.kernel(
    _sc_deg_body,
    out_type=jax.ShapeDtypeStruct((NP, HH), jnp.float32),
    mesh=_MESH,
    compiler_params=_SC_PARAMS,
    scratch_types=[
        pltpu.VMEM((NR, KB2 * CHUNK), jnp.int32),
        pltpu.VMEM((KB2 * CHUNK, DEGW), jnp.float32),
        pltpu.VMEM_SHARED((NP, DEGW), jnp.float32),
    ],
)


def _sc_gs_body(ht, eidx, zeros_c, out, *scratch):
    sidx, didx = scratch[0], scratch[1]
    rows = scratch[2:4]
    tsh = scratch[4]
    acc = scratch[5]
    gsems = scratch[6:8]
    ssems = scratch[8:10]
    isems = scratch[10:14]
    cid = lax.axis_index("c")
    sid = lax.axis_index("s")

    pltpu.sync_copy(zeros_c, acc.at[pl.ds(sid * RPS, RPS)])

    @pl.when(cid == 0)
    def _():
        pltpu.sync_copy(ht.at[pl.ds(sid * RPS, RPS), pl.ds(0, HC)],
                        tsh.at[pl.ds(sid * RPS, RPS)])

    @pl.when(cid == 1)
    def _():
        pltpu.sync_copy(ht.at[pl.ds(sid * RPS, RPS), pl.ds(HC, HC)],
                        tsh.at[pl.ds(sid * RPS, RPS)])

    plsc.subcore_barrier()

    def prefetch(r, slot):
        pltpu.async_copy(eidx.at[0, sid, r], sidx.at[slot], isems[slot])
        pltpu.async_copy(eidx.at[1, sid, r], didx.at[slot], isems[slot])

    def wait_idx(r, slot):
        pltpu.make_async_copy(eidx.at[0, sid, r], sidx.at[slot],
                              isems[slot]).wait()
        pltpu.make_async_copy(eidx.at[1, sid, r], didx.at[slot],
                              isems[slot]).wait()

    prefetch(0, 0)
    prefetch(1, 1)

    def body(q, c):
        for rr in range(4):
            r = q * 4 + rr
            p = rr % 2
            pslot = (rr + 2) % 4
            @pl.when(r >= 2)
            def _():
                pltpu.make_async_copy(rows[p], acc.at[didx.at[pslot]],
                                      ssems[p]).wait()

            @pl.when(r + 2 < NR)
            def _():
                prefetch(r + 2, pslot)

            wait_idx(r, rr)
            pltpu.async_copy(tsh.at[sidx.at[rr]], rows[p], gsems[p])
            pltpu.make_async_copy(tsh.at[sidx.at[rr]], rows[p],
                                  gsems[p]).wait()
            pltpu.async_copy(rows[p], acc.at[didx.at[rr]], ssems[p],
                             add=True)
        return c

    lax.fori_loop(0, NR // 4, body, 0)
    for rr in (NR - 2) % 4, (NR - 1) % 4:
        p = rr % 2
        pltpu.make_async_copy(rows[p], acc.at[didx.at[rr]],
                              ssems[p]).wait()
    plsc.subcore_barrier()

    @pl.when(cid == 0)
    def _():
        pltpu.sync_copy(acc.at[pl.ds(sid * RPS, RPS)],
                        out.at[pl.ds(sid * RPS, RPS), pl.ds(0, HC)])

    @pl.when(cid == 1)
    def _():
        pltpu.sync_copy(acc.at[pl.ds(sid * RPS, RPS)],
                        out.at[pl.ds(sid * RPS, RPS), pl.ds(HC, HC)])


_sc_gs = pl.kernel(
    _sc_gs_body,
    out_type=jax.ShapeDtypeStruct((NP, HH), jnp.float32),
    mesh=_MESH,
    compiler_params=_SC_PARAMS,
    scratch_types=(
        [pltpu.VMEM((4, KB2 * CHUNK), jnp.int32),
         pltpu.VMEM((4, KB2 * CHUNK), jnp.int32)]
        + [pltpu.VMEM((KB2 * CHUNK, HC), jnp.float32) for _ in range(2)]
        + [pltpu.VMEM_SHARED((NP, HC), jnp.float32),
           pltpu.VMEM_SHARED((NP, HC), jnp.float32)]
        + [pltpu.SemaphoreType.DMA for _ in range(8)]
    ),
)


def _tc_pad_body(ei_ref, out_ref):
    out_ref[:, :EE] = ei_ref[...]
    out_ref[:, EE:] = jnp.full((2, EP - EE), NN, jnp.int32)


_tc_pad = pl.pallas_call(
    _tc_pad_body,
    out_shape=jax.ShapeDtypeStruct((2, EP), jnp.int32),
)


def _dinv_block(deg_ref):
    return lax.rsqrt(1.0 + deg_ref[:, 0:1] + deg_ref[:, DEGW:DEGW + 1])


def _tc_m_body(x_ref, w0_ref, b0_ref, w1_ref, b1_ref, out_ref):
    h0 = jnp.dot(x_ref[...], w0_ref[...], precision=_PREC,
                 preferred_element_type=jnp.float32) + b0_ref[...]
    out_ref[...] = (jnp.dot(h0, w1_ref[...], precision=_PREC,
                            preferred_element_type=jnp.float32)
                    + b1_ref[...])


def _tc_scale_body(h_ref, deg_ref, out_ref):
    out_ref[...] = _dinv_block(deg_ref) * h_ref[...]


def _tc_b_body(s_ref, ht_ref, deg_ref, w_ref, b_ref, out_ref):
    dinv = _dinv_block(deg_ref)
    t = jnp.maximum(dinv * (s_ref[...] + ht_ref[...]), 0.0)
    out_ref[...] = dinv * (jnp.dot(t, w_ref[...], precision=_PREC,
                                   preferred_element_type=jnp.float32)
                           + b_ref[...])


def _tc_c_body(s_ref, ht_ref, deg_ref, batch_ref, out_ref):
    dinv = _dinv_block(deg_ref)
    agg = jnp.where(batch_ref[...] < GG,
                    dinv * (s_ref[...] + ht_ref[...]), 0.0)
    onehot = (batch_ref[...] ==
              lax.broadcasted_iota(jnp.int32, (BM, GG), 1)).astype(jnp.float32)
    contrib = lax.dot_general(onehot, agg, (((0,), (0,)), ((), ())),
                              precision=_PREC,
                              preferred_element_type=jnp.float32)

    @pl.when(pl.program_id(0) == 0)
    def _():
        out_ref[...] = jnp.zeros_like(out_ref)

    out_ref[...] += contrib


_rows_spec = pl.BlockSpec((BM, HH), lambda i: (i, 0))
_x_spec = pl.BlockSpec((BM, DIN), lambda i: (i, 0))
_w0_spec = pl.BlockSpec((DIN, HH), lambda i: (0, 0))
_w_spec = pl.BlockSpec((HH, HH), lambda i: (0, 0))
_b_spec = pl.BlockSpec((1, HH), lambda i: (0, 0))
_batch_spec = pl.BlockSpec((BM, 1), lambda i: (i, 0))
_rows_shape = jax.ShapeDtypeStruct((NP, HH), jnp.float32)

_tc_m = pl.pallas_call(
    _tc_m_body,
    grid=(GRID,),
    in_specs=[_x_spec, _w0_spec, _b_spec, _w_spec, _b_spec],
    out_specs=_rows_spec,
    out_shape=_rows_shape,
)

_tc_scale = pl.pallas_call(
    _tc_scale_body,
    grid=(GRID,),
    in_specs=[_rows_spec, _rows_spec],
    out_specs=_rows_spec,
    out_shape=_rows_shape,
)

_tc_b = pl.pallas_call(
    _tc_b_body,
    grid=(GRID,),
    in_specs=[_rows_spec, _rows_spec, _rows_spec, _w_spec, _b_spec],
    out_specs=_rows_spec,
    out_shape=_rows_shape,
)

_tc_c = pl.pallas_call(
    _tc_c_body,
    grid=(GRID,),
    in_specs=[_rows_spec, _rows_spec, _rows_spec, _batch_spec],
    out_specs=pl.BlockSpec((GG, HH), lambda i: (0, 0)),
    out_shape=jax.ShapeDtypeStruct((GG, HH), jnp.float32),
)


def kernel(x, edge_index, batch, W0, b0, W1, b1, W2, b2, W3, b3):
    f32 = jnp.float32
    b0r = b0.reshape(1, HH)
    b1r = b1.reshape(1, HH)
    b2r = b2.reshape(1, HH)
    b3r = b3.reshape(1, HH)

    e4 = _tc_pad(edge_index).reshape(2, NS, NR, KB2 * CHUNK)
    batchp = jnp.concatenate(
        [batch, jnp.full((NP - NN,), GG, jnp.int32)]).reshape(NP, 1)

    zeros_c = jnp.zeros((RPS, HC), f32)
    zeros_d = jnp.zeros((RPS, DEGW), f32)
    ones_d = jnp.ones((KB2 * CHUNK, DEGW), f32)

    deg2 = _sc_deg(e4, zeros_d, ones_d)
    hh1 = _tc_m(x, W0, b0r, W1, b1r)
    ht1 = _tc_scale(hh1, deg2)
    s1 = _sc_gs(ht1, e4, zeros_c)
    ht2 = _tc_b(s1, ht1, deg2, W2, b2r)
    s2 = _sc_gs(ht2, e4, zeros_c)
    ht3 = _tc_b(s2, ht2, deg2, W3, b3r)
    s3 = _sc_gs(ht3, e4, zeros_c)
    pooled = _tc_c(s3, ht3, deg2, batchp)
    return pooled
